# baseline (speedup 1.0000x reference)
import jax
import jax.numpy as jnp
from jax import lax
from jax.experimental import pallas as pl
from jax.experimental.pallas import tpu as pltpu
from jax.experimental.pallas import tpu_sc as plsc

BATCH = 16384
NROWS = 1000000
EMBED_K = 16
NC = 2
NS = 16
L = 16
NW = NC * NS
B_PER_W = BATCH // NW
N_GROUPS = B_PER_W // L
IDX_CHUNK = 128
N_IDX_CHUNKS = B_PER_W // IDX_CHUNK


def _mf_body(x_hbm, wt_hbm, ht_hbm, out_hbm,
             xbuf, uidx, vidx, ubuf, vbuf, outv, usem, vsem):
    wid = lax.axis_index("s") * NC + lax.axis_index("c")
    base = pl.multiple_of(wid * B_PER_W, B_PER_W)

    pltpu.sync_copy(x_hbm.at[pl.ds(base * 2, 2 * B_PER_W)], xbuf)
    lane = lax.iota(jnp.int32, L)

    def extract(g, _):
        pair = jnp.full((L,), 2 * g * L, jnp.int32) + 2 * lane
        off = pl.ds(pl.multiple_of(g * L, L), L)
        uidx[off] = plsc.load_gather(xbuf, [pair])
        vidx[off] = plsc.load_gather(xbuf, [pair + 1])
        return 0

    lax.fori_loop(0, N_GROUPS, extract, 0, unroll=4)

    for k in range(EMBED_K):
        wseg = wt_hbm.at[pl.ds(k * NROWS, NROWS)]
        hseg = ht_hbm.at[pl.ds(k * NROWS, NROWS)]
        for j in range(N_IDX_CHUNKS):
            isl = pl.ds(j * IDX_CHUNK, IDX_CHUNK)
            dsl = pl.ds(k * B_PER_W + j * IDX_CHUNK, IDX_CHUNK)
            pltpu.async_copy(wseg.at[uidx.at[isl]], ubuf.at[dsl], usem)
            pltpu.async_copy(hseg.at[vidx.at[isl]], vbuf.at[dsl], vsem)

    pltpu.make_async_copy(
        wt_hbm.at[pl.ds(0, EMBED_K * B_PER_W)], ubuf, usem).wait()
    pltpu.make_async_copy(
        ht_hbm.at[pl.ds(0, EMBED_K * B_PER_W)], vbuf, vsem).wait()

    def compute(g, _):
        off = pl.multiple_of(g * L, L)
        acc = ubuf[pl.ds(off, L)] * vbuf[pl.ds(off, L)]
        for k in range(1, EMBED_K):
            sl = pl.ds(pl.multiple_of(k * B_PER_W + g * L, L), L)
            acc += ubuf[sl] * vbuf[sl]
        outv[pl.ds(off, L)] = acc
        return 0

    lax.fori_loop(0, N_GROUPS, compute, 0, unroll=2)

    pltpu.sync_copy(outv, out_hbm.at[pl.ds(base, B_PER_W)])


@jax.jit
def _mf_kernel(x, Wt, Ht):
    mesh = plsc.VectorSubcoreMesh(core_axis_name="c", subcore_axis_name="s")
    return pl.kernel(
        _mf_body,
        out_type=jax.ShapeDtypeStruct((BATCH,), jnp.float32),
        mesh=mesh,
        compiler_params=pltpu.CompilerParams(
            needs_layout_passes=False, use_tc_tiling_on_sc=False),
        scratch_types=[
            pltpu.VMEM((2 * B_PER_W,), jnp.int32),
            pltpu.VMEM((B_PER_W,), jnp.int32),
            pltpu.VMEM((B_PER_W,), jnp.int32),
            pltpu.VMEM((EMBED_K * B_PER_W,), jnp.float32),
            pltpu.VMEM((EMBED_K * B_PER_W,), jnp.float32),
            pltpu.VMEM((B_PER_W,), jnp.float32),
            pltpu.SemaphoreType.DMA,
            pltpu.SemaphoreType.DMA,
        ],
    )(x, Wt, Ht)


def kernel(x, W, H):
    return _mf_kernel(x.reshape(-1), W.T.reshape(-1), H.T.reshape(-1))


# speedup vs baseline: 17.4118x; 17.4118x over previous
import jax
import jax.numpy as jnp
from jax import lax
from jax.experimental import pallas as pl
from jax.experimental.pallas import tpu as pltpu
from jax.experimental.pallas import tpu_sc as plsc

BATCH = 16384
NROWS = 1000000
EMBED_K = 16
NC = 2
NS = 16
L = 16
NW = NC * NS
B_PER_W = BATCH // NW
CHB = 16
N_CHUNKS = B_PER_W // CHB


def _mf_body(x_hbm, wt_hbm, ht_hbm, out_hbm,
             xbuf, ublk, vblk, outv, usem, vsem):
    wid = lax.axis_index("s") * NC + lax.axis_index("c")
    base = pl.multiple_of(wid * B_PER_W, B_PER_W)

    pltpu.sync_copy(x_hbm.at[pl.ds(base * 2, 2 * B_PER_W)], xbuf)

    lane = lax.iota(jnp.int32, L)
    slot = lane

    def chunk(c, _):
        pair = jnp.full((L,), 2 * c * CHB, jnp.int32) + 2 * lane
        uvec = plsc.load_gather(xbuf, [pair])
        vvec = plsc.load_gather(xbuf, [pair + 1])

        for j in range(CHB):
            uoff = pl.multiple_of((uvec[j] >> 7) * 128, 128)
            voff = pl.multiple_of((vvec[j] >> 7) * 128, 128)
            pltpu.async_copy(wt_hbm.at[:, pl.ds(uoff, 128)],
                             ublk.at[j], usem)
            pltpu.async_copy(ht_hbm.at[:, pl.ds(voff, 128)],
                             vblk.at[j], vsem)

        for j in range(CHB):
            pltpu.make_async_copy(
                wt_hbm.at[:, pl.ds(0, 128)], ublk.at[0], usem).wait()
            pltpu.make_async_copy(
                ht_hbm.at[:, pl.ds(0, 128)], vblk.at[0], vsem).wait()

        ulane = uvec & 127
        vlane = vvec & 127
        acc = plsc.load_gather(ublk, [slot, jnp.zeros((L,), jnp.int32), ulane]) * \
              plsc.load_gather(vblk, [slot, jnp.zeros((L,), jnp.int32), vlane])
        for k in range(1, EMBED_K):
            ck = jnp.full((L,), k, jnp.int32)
            acc += plsc.load_gather(ublk, [slot, ck, ulane]) * \
                   plsc.load_gather(vblk, [slot, ck, vlane])
        outv[pl.ds(pl.multiple_of(c * CHB, CHB), CHB)] = acc
        return 0

    lax.fori_loop(0, N_CHUNKS, chunk, 0)

    pltpu.sync_copy(outv, out_hbm.at[pl.ds(base, B_PER_W)])


@jax.jit
def _mf_kernel(x, Wt, Ht):
    mesh = plsc.VectorSubcoreMesh(core_axis_name="c", subcore_axis_name="s")
    return pl.kernel(
        _mf_body,
        out_type=jax.ShapeDtypeStruct((BATCH,), jnp.float32),
        mesh=mesh,
        compiler_params=pltpu.CompilerParams(needs_layout_passes=False),
        scratch_types=[
            pltpu.VMEM((2 * B_PER_W,), jnp.int32),
            pltpu.VMEM((CHB, EMBED_K, 128), jnp.float32),
            pltpu.VMEM((CHB, EMBED_K, 128), jnp.float32),
            pltpu.VMEM((B_PER_W,), jnp.float32),
            pltpu.SemaphoreType.DMA,
            pltpu.SemaphoreType.DMA,
        ],
    )(x, Wt, Ht)


def kernel(x, W, H):
    return _mf_kernel(x.reshape(-1), W.T, H.T)


# speedup vs baseline: 17.4290x; 1.0010x over previous
import jax
import jax.numpy as jnp
from jax import lax
from jax.experimental import pallas as pl
from jax.experimental.pallas import tpu as pltpu
from jax.experimental.pallas import tpu_sc as plsc

BATCH = 16384
NROWS = 1000000
EMBED_K = 16
NC = 2
NS = 16
L = 16
NW = NC * NS
B_PER_W = BATCH // NW
CHB = 16
N_CHUNKS = B_PER_W // CHB


def _mf_body(x_hbm, wt_hbm, ht_hbm, out_hbm,
             xbuf, ublk, vblk, outv, usem, vsem):
    wid = lax.axis_index("s") * NC + lax.axis_index("c")
    base = pl.multiple_of(wid * B_PER_W, B_PER_W)

    pltpu.sync_copy(x_hbm.at[pl.ds(base * 2, 2 * B_PER_W)], xbuf)

    lane = lax.iota(jnp.int32, L)
    slot = lane

    def chunk(c, _):
        pair = jnp.full((L,), 2 * c * CHB, jnp.int32) + 2 * lane
        uvec = plsc.load_gather(xbuf, [pair])
        vvec = plsc.load_gather(xbuf, [pair + 1])

        for j in range(CHB):
            uoff = pl.multiple_of((uvec[j] >> 7) * 128, 128)
            voff = pl.multiple_of((vvec[j] >> 7) * 128, 128)
            pltpu.async_copy(wt_hbm.at[:, pl.ds(uoff, 128)],
                             ublk.at[j], usem)
            pltpu.async_copy(ht_hbm.at[:, pl.ds(voff, 128)],
                             vblk.at[j], vsem)

        for j in range(CHB):
            pltpu.make_async_copy(
                wt_hbm.at[:, pl.ds(0, 128)], ublk.at[0], usem).wait()
            pltpu.make_async_copy(
                ht_hbm.at[:, pl.ds(0, 128)], vblk.at[0], vsem).wait()

        ulane = uvec & 127
        vlane = vvec & 127
        acc = plsc.load_gather(ublk, [slot, jnp.zeros((L,), jnp.int32), ulane]) * \
              plsc.load_gather(vblk, [slot, jnp.zeros((L,), jnp.int32), vlane])
        for k in range(1, EMBED_K):
            ck = jnp.full((L,), k, jnp.int32)
            acc += plsc.load_gather(ublk, [slot, ck, ulane]) * \
                   plsc.load_gather(vblk, [slot, ck, vlane])
        outv[pl.ds(pl.multiple_of(c * CHB, CHB), CHB)] = acc
        return 0

    lax.fori_loop(0, N_CHUNKS, chunk, 0, unroll=2)

    pltpu.sync_copy(outv, out_hbm.at[pl.ds(base, B_PER_W)])


@jax.jit
def _mf_kernel(x, Wt, Ht):
    mesh = plsc.VectorSubcoreMesh(core_axis_name="c", subcore_axis_name="s")
    return pl.kernel(
        _mf_body,
        out_type=jax.ShapeDtypeStruct((BATCH,), jnp.float32),
        mesh=mesh,
        compiler_params=pltpu.CompilerParams(needs_layout_passes=False),
        scratch_types=[
            pltpu.VMEM((2 * B_PER_W,), jnp.int32),
            pltpu.VMEM((CHB, EMBED_K, 128), jnp.float32),
            pltpu.VMEM((CHB, EMBED_K, 128), jnp.float32),
            pltpu.VMEM((B_PER_W,), jnp.float32),
            pltpu.SemaphoreType.DMA,
            pltpu.SemaphoreType.DMA,
        ],
    )(x, Wt, Ht)


def kernel(x, W, H):
    return _mf_kernel(x.reshape(-1), W.T, H.T)
